# Initial kernel scaffold; baseline (speedup 1.0000x reference)
import jax
import jax.numpy as jnp
from jax.experimental import pallas as pl


def kernel(feats, adj_stack, edge_stack, w_enc, weight_t, bias):
    raise NotImplementedError("write your pallas kernel here")


# speedup vs baseline: 1.0958x; 1.0958x over previous
import jax
import jax.numpy as jnp
from jax.experimental import pallas as pl
from jax.experimental.pallas import tpu as pltpu


def _round_up(x, m):
    return ((x + m - 1) // m) * m


def _encoder_scores_kernel(adj_ref, fh_ref, fl_ref, wenc_ref, w2t_ref, bias_ref,
                           s0_ref, s1_ref):
    a = adj_ref[...].astype(jnp.bfloat16)
    p = (jnp.dot(a, fh_ref[...], preferred_element_type=jnp.float32)
         + jnp.dot(a, fl_ref[...], preferred_element_type=jnp.float32))
    emb = jnp.tanh(jnp.dot(p, wenc_ref[...],
                           preferred_element_type=jnp.float32))
    sc = jax.lax.dot_general(
        w2t_ref[...], emb, dimension_numbers=(((1,), (1,)), ((), ())),
        preferred_element_type=jnp.float32) + bias_ref[...]
    s0_ref[...] = sc[0:1, :]
    s1_ref[...] = sc[1:2, :]


def _edge_logits_kernel(src_ref, dst_ref, s0_ref, s1_ref, out_ref):
    te = src_ref.shape[-1]
    nq = s0_ref.shape[0]
    r_iota = jax.lax.broadcasted_iota(jnp.int32, (128, te), 0)
    q_iota = jax.lax.broadcasted_iota(jnp.int32, (nq, te), 0)

    def pick(tab, idx):
        r = jnp.bitwise_and(idx, 127)
        q = jnp.right_shift(idx, 7)
        ohr = (r_iota == r).astype(jnp.float32)
        u = jnp.dot(tab, ohr, preferred_element_type=jnp.float32)
        return jnp.sum(jnp.where(q_iota == q, u, 0.0), axis=0,
                       keepdims=True)

    out_ref[...] = (pick(s0_ref[...], src_ref[...])
                    + pick(s1_ref[...], dst_ref[...]))


def kernel(feats, adj_stack, edge_stack, w_enc, weight_t, bias):
    n_views, n_nodes, _ = adj_stack.shape
    f = feats.shape[1]
    d = w_enc.shape[1]
    n_edges = edge_stack.shape[2]
    nq = n_nodes // 128

    w2t = jnp.concatenate([weight_t[:d, :].T, weight_t[d:, :].T],
                          axis=0).astype(jnp.float32)
    bias2 = jnp.concatenate(
        [bias.reshape(1, 1).astype(jnp.float32),
         jnp.zeros((1, 1), jnp.float32)], axis=0)

    fh = feats.astype(jnp.bfloat16)
    fl = (feats - fh.astype(jnp.float32)).astype(jnp.bfloat16)

    tm = 128
    s0, s1 = pl.pallas_call(
        _encoder_scores_kernel,
        out_shape=(jax.ShapeDtypeStruct((n_views, nq, 1, 128), jnp.float32),
                   jax.ShapeDtypeStruct((n_views, nq, 1, 128), jnp.float32)),
        grid=(n_views, n_nodes // tm),
        in_specs=[
            pl.BlockSpec((None, tm, n_nodes), lambda vi, ni: (vi, ni, 0)),
            pl.BlockSpec((n_nodes, f), lambda vi, ni: (0, 0)),
            pl.BlockSpec((n_nodes, f), lambda vi, ni: (0, 0)),
            pl.BlockSpec((f, d), lambda vi, ni: (0, 0)),
            pl.BlockSpec((2, d), lambda vi, ni: (0, 0)),
            pl.BlockSpec((2, 1), lambda vi, ni: (0, 0)),
        ],
        out_specs=(pl.BlockSpec((None, None, 1, 128),
                                lambda vi, ni: (vi, ni, 0, 0)),
                   pl.BlockSpec((None, None, 1, 128),
                                lambda vi, ni: (vi, ni, 0, 0))),
        compiler_params=pltpu.CompilerParams(
            dimension_semantics=("parallel", "parallel"),
            vmem_limit_bytes=64 * 1024 * 1024),
    )(adj_stack, fh, fl, w_enc.astype(jnp.float32), w2t, bias2)
    s0 = s0.reshape(n_views, nq, 128)
    s1 = s1.reshape(n_views, nq, 128)

    te = 2048 if n_edges >= 2048 else _round_up(n_edges, 128)
    n_et = pl.cdiv(n_edges, te)
    e_pad = n_et * te
    src_p = jnp.pad(edge_stack[:, 0, :].astype(jnp.int32),
                    ((0, 0), (0, e_pad - n_edges))).reshape(n_views, 1, e_pad)
    dst_p = jnp.pad(edge_stack[:, 1, :].astype(jnp.int32),
                    ((0, 0), (0, e_pad - n_edges))).reshape(n_views, 1, e_pad)

    out = pl.pallas_call(
        _edge_logits_kernel,
        out_shape=jax.ShapeDtypeStruct((n_views, 1, e_pad), jnp.float32),
        grid=(n_views, n_et),
        in_specs=[
            pl.BlockSpec((None, 1, te), lambda vi, ei: (vi, 0, ei)),
            pl.BlockSpec((None, 1, te), lambda vi, ei: (vi, 0, ei)),
            pl.BlockSpec((None, nq, 128), lambda vi, ei: (vi, 0, 0)),
            pl.BlockSpec((None, nq, 128), lambda vi, ei: (vi, 0, 0)),
        ],
        out_specs=pl.BlockSpec((None, 1, te), lambda vi, ei: (vi, 0, ei)),
        compiler_params=pltpu.CompilerParams(
            dimension_semantics=("parallel", "parallel"),
            vmem_limit_bytes=64 * 1024 * 1024),
    )(src_p, dst_p, s0, s1)

    logits = out[:, 0, :n_edges][..., None]
    return [logits[i] for i in range(n_views)]


# speedup vs baseline: 2.0141x; 1.8381x over previous
import jax
import jax.numpy as jnp
from jax.experimental import pallas as pl
from jax.experimental.pallas import tpu as pltpu


def _round_up(x, m):
    return ((x + m - 1) // m) * m


def _encoder_scores_kernel(adj_ref, fh_ref, fl_ref, wenc_ref, w2t_ref, bias_ref,
                           s_ref):
    a = adj_ref[...].astype(jnp.bfloat16)
    p = (jnp.dot(a, fh_ref[...], preferred_element_type=jnp.float32)
         + jnp.dot(a, fl_ref[...], preferred_element_type=jnp.float32))
    emb = jnp.tanh(jnp.dot(p, wenc_ref[...],
                           preferred_element_type=jnp.float32))
    s_ref[...] = jax.lax.dot_general(
        w2t_ref[...], emb, dimension_numbers=(((1,), (1,)), ((), ())),
        preferred_element_type=jnp.float32) + bias_ref[...]


def _edge_logits_kernel(edges_ref, s_ref, out_ref):
    te = edges_ref.shape[-1]
    s_all = s_ref[...]
    n = s_all.shape[1]
    nq = n // 128
    t0 = s_all[0:1, :].reshape(nq, 128)
    t1 = s_all[1:2, :].reshape(nq, 128)
    r_iota = jax.lax.broadcasted_iota(jnp.int32, (128, te), 0)
    q_iota = jax.lax.broadcasted_iota(jnp.int32, (nq, te), 0)

    def pick(tab, idx):
        r = jnp.bitwise_and(idx, 127)
        q = jnp.right_shift(idx, 7)
        ohr = (r_iota == r).astype(jnp.float32)
        u = jnp.dot(tab, ohr, preferred_element_type=jnp.float32)
        return jnp.sum(jnp.where(q_iota == q, u, 0.0), axis=0,
                       keepdims=True)

    out_ref[...] = (pick(t0, edges_ref[0:1, :])
                    + pick(t1, edges_ref[1:2, :]))


def kernel(feats, adj_stack, edge_stack, w_enc, weight_t, bias):
    n_views, n_nodes, _ = adj_stack.shape
    f = feats.shape[1]
    d = w_enc.shape[1]
    n_edges = edge_stack.shape[2]

    w2t = jnp.concatenate([weight_t[:d, :].T, weight_t[d:, :].T],
                          axis=0).astype(jnp.float32)
    bias2 = jnp.concatenate(
        [bias.reshape(1, 1).astype(jnp.float32),
         jnp.zeros((1, 1), jnp.float32)], axis=0)

    fh = feats.astype(jnp.bfloat16)
    fl = (feats - fh.astype(jnp.float32)).astype(jnp.bfloat16)

    tm = min(512, n_nodes)
    scores = pl.pallas_call(
        _encoder_scores_kernel,
        out_shape=jax.ShapeDtypeStruct((n_views, 2, n_nodes), jnp.float32),
        grid=(n_views, n_nodes // tm),
        in_specs=[
            pl.BlockSpec((None, tm, n_nodes), lambda vi, ni: (vi, ni, 0)),
            pl.BlockSpec((n_nodes, f), lambda vi, ni: (0, 0)),
            pl.BlockSpec((n_nodes, f), lambda vi, ni: (0, 0)),
            pl.BlockSpec((f, d), lambda vi, ni: (0, 0)),
            pl.BlockSpec((2, d), lambda vi, ni: (0, 0)),
            pl.BlockSpec((2, 1), lambda vi, ni: (0, 0)),
        ],
        out_specs=pl.BlockSpec((None, 2, tm), lambda vi, ni: (vi, 0, ni)),
        compiler_params=pltpu.CompilerParams(
            dimension_semantics=("parallel", "parallel"),
            vmem_limit_bytes=100 * 1024 * 1024),
    )(adj_stack, fh, fl, w_enc.astype(jnp.float32), w2t, bias2)

    e_pad = _round_up(n_edges, 128)
    edges_p = jnp.pad(edge_stack.astype(jnp.int32),
                      ((0, 0), (0, 0), (0, e_pad - n_edges)))

    out = pl.pallas_call(
        _edge_logits_kernel,
        out_shape=jax.ShapeDtypeStruct((n_views, 1, e_pad), jnp.float32),
        grid=(n_views,),
        in_specs=[
            pl.BlockSpec((None, 2, e_pad), lambda vi: (vi, 0, 0)),
            pl.BlockSpec((None, 2, n_nodes), lambda vi: (vi, 0, 0)),
        ],
        out_specs=pl.BlockSpec((None, 1, e_pad), lambda vi: (vi, 0, 0)),
        compiler_params=pltpu.CompilerParams(
            dimension_semantics=("parallel",),
            vmem_limit_bytes=100 * 1024 * 1024),
    )(edges_p, scores)

    logits = out[:, 0, :n_edges][..., None]
    return [logits[i] for i in range(n_views)]


# speedup vs baseline: 2.2244x; 1.1044x over previous
import jax
import jax.numpy as jnp
from jax.experimental import pallas as pl
from jax.experimental.pallas import tpu as pltpu


def _round_up(x, m):
    return ((x + m - 1) // m) * m


def _encoder_scores_kernel(adj_ref, fh_ref, fl_ref, wenc_ref, w2t_ref, bias_ref,
                           s_ref):
    a = adj_ref[...].astype(jnp.bfloat16)
    p = (jnp.dot(a, fh_ref[...], preferred_element_type=jnp.float32)
         + jnp.dot(a, fl_ref[...], preferred_element_type=jnp.float32))
    emb = jnp.tanh(jnp.dot(p, wenc_ref[...],
                           preferred_element_type=jnp.float32))
    s_ref[...] = jax.lax.dot_general(
        w2t_ref[...], emb, dimension_numbers=(((1,), (1,)), ((), ())),
        preferred_element_type=jnp.float32) + bias_ref[...]


def _edge_logits_kernel(edges_ref, s_ref, out_ref):
    te = edges_ref.shape[-1]
    s_all = s_ref[...]
    n = s_all.shape[1]
    nq = n // 128
    t0 = s_all[0:1, :].reshape(nq, 128)
    t1 = s_all[1:2, :].reshape(nq, 128)
    r_iota = jax.lax.broadcasted_iota(jnp.int32, (128, te), 0)
    q_iota = jax.lax.broadcasted_iota(jnp.int32, (nq, te), 0)

    def pick(tab, idx):
        r = jnp.bitwise_and(idx, 127)
        q = jnp.right_shift(idx, 7)
        ohr = (r_iota == r).astype(jnp.float32)
        u = jnp.dot(tab, ohr, preferred_element_type=jnp.float32)
        return jnp.sum(jnp.where(q_iota == q, u, 0.0), axis=0,
                       keepdims=True)

    out_ref[...] = (pick(t0, edges_ref[0:1, :])
                    + pick(t1, edges_ref[1:2, :]))


def kernel(feats, adj_stack, edge_stack, w_enc, weight_t, bias):
    n_views, n_nodes, _ = adj_stack.shape
    f = feats.shape[1]
    d = w_enc.shape[1]
    n_edges = edge_stack.shape[2]

    w2t = jnp.concatenate([weight_t[:d, :].T, weight_t[d:, :].T],
                          axis=0).astype(jnp.float32)
    bias2 = jnp.concatenate(
        [bias.reshape(1, 1).astype(jnp.float32),
         jnp.zeros((1, 1), jnp.float32)], axis=0)

    fh = feats.astype(jnp.bfloat16)
    fl = (feats - fh.astype(jnp.float32)).astype(jnp.bfloat16)

    tm = min(1024, n_nodes)
    scores = pl.pallas_call(
        _encoder_scores_kernel,
        out_shape=jax.ShapeDtypeStruct((n_views, 2, n_nodes), jnp.float32),
        grid=(n_views, n_nodes // tm),
        in_specs=[
            pl.BlockSpec((None, tm, n_nodes), lambda vi, ni: (vi, ni, 0)),
            pl.BlockSpec((n_nodes, f), lambda vi, ni: (0, 0)),
            pl.BlockSpec((n_nodes, f), lambda vi, ni: (0, 0)),
            pl.BlockSpec((f, d), lambda vi, ni: (0, 0)),
            pl.BlockSpec((2, d), lambda vi, ni: (0, 0)),
            pl.BlockSpec((2, 1), lambda vi, ni: (0, 0)),
        ],
        out_specs=pl.BlockSpec((None, 2, tm), lambda vi, ni: (vi, 0, ni)),
        compiler_params=pltpu.CompilerParams(
            dimension_semantics=("parallel", "parallel"),
            vmem_limit_bytes=100 * 1024 * 1024),
    )(adj_stack, fh, fl, w_enc.astype(jnp.float32), w2t, bias2)

    e_pad = _round_up(n_edges, 128)
    edges_p = jnp.pad(edge_stack.astype(jnp.int32),
                      ((0, 0), (0, 0), (0, e_pad - n_edges)))

    out = pl.pallas_call(
        _edge_logits_kernel,
        out_shape=jax.ShapeDtypeStruct((n_views, 1, e_pad), jnp.float32),
        grid=(n_views,),
        in_specs=[
            pl.BlockSpec((None, 2, e_pad), lambda vi: (vi, 0, 0)),
            pl.BlockSpec((None, 2, n_nodes), lambda vi: (vi, 0, 0)),
        ],
        out_specs=pl.BlockSpec((None, 1, e_pad), lambda vi: (vi, 0, 0)),
        compiler_params=pltpu.CompilerParams(
            dimension_semantics=("parallel",),
            vmem_limit_bytes=100 * 1024 * 1024),
    )(edges_p, scores)

    logits = out[:, 0, :n_edges][..., None]
    return [logits[i] for i in range(n_views)]


# speedup vs baseline: 2.2338x; 1.0042x over previous
import jax
import jax.numpy as jnp
from jax.experimental import pallas as pl
from jax.experimental.pallas import tpu as pltpu


def _round_up(x, m):
    return ((x + m - 1) // m) * m


def _encoder_scores_kernel(adj_ref, fh_ref, fl_ref, wenc_ref, w2t_ref, bias_ref,
                           s_ref):
    a = adj_ref[...].astype(jnp.bfloat16)
    p = (jnp.dot(a, fh_ref[...], preferred_element_type=jnp.float32)
         + jnp.dot(a, fl_ref[...], preferred_element_type=jnp.float32))
    emb = jnp.tanh(jnp.dot(p, wenc_ref[...],
                           preferred_element_type=jnp.float32))
    s_ref[...] = jax.lax.dot_general(
        w2t_ref[...], emb, dimension_numbers=(((1,), (1,)), ((), ())),
        preferred_element_type=jnp.float32) + bias_ref[...]


def _edge_logits_kernel(edges_ref, s_ref, out_ref):
    te = edges_ref.shape[-1]
    s_all = s_ref[...]
    n = s_all.shape[1]
    nq = n // 128
    t0 = s_all[0:1, :].reshape(nq, 128)
    t1 = s_all[1:2, :].reshape(nq, 128)
    r_iota = jax.lax.broadcasted_iota(jnp.int32, (128, te), 0)
    q_iota = jax.lax.broadcasted_iota(jnp.int32, (nq, te), 0)

    def pick(tab, idx):
        r = jnp.bitwise_and(idx, 127)
        q = jnp.right_shift(idx, 7)
        ohr = (r_iota == r).astype(jnp.float32)
        u = jnp.dot(tab, ohr, preferred_element_type=jnp.float32)
        return jnp.sum(jnp.where(q_iota == q, u, 0.0), axis=0,
                       keepdims=True)

    out_ref[...] = (pick(t0, edges_ref[0:1, :])
                    + pick(t1, edges_ref[1:2, :]))


def kernel(feats, adj_stack, edge_stack, w_enc, weight_t, bias):
    n_views, n_nodes, _ = adj_stack.shape
    f = feats.shape[1]
    d = w_enc.shape[1]
    n_edges = edge_stack.shape[2]

    w2t = jnp.concatenate([weight_t[:d, :].T, weight_t[d:, :].T],
                          axis=0).astype(jnp.float32)
    bias2 = jnp.concatenate(
        [bias.reshape(1, 1).astype(jnp.float32),
         jnp.zeros((1, 1), jnp.float32)], axis=0)

    fh = feats.astype(jnp.bfloat16)
    fl = (feats - fh.astype(jnp.float32)).astype(jnp.bfloat16)

    tm = min(2048, n_nodes)
    scores = pl.pallas_call(
        _encoder_scores_kernel,
        out_shape=jax.ShapeDtypeStruct((n_views, 2, n_nodes), jnp.float32),
        grid=(n_views, n_nodes // tm),
        in_specs=[
            pl.BlockSpec((None, tm, n_nodes), lambda vi, ni: (vi, ni, 0)),
            pl.BlockSpec((n_nodes, f), lambda vi, ni: (0, 0)),
            pl.BlockSpec((n_nodes, f), lambda vi, ni: (0, 0)),
            pl.BlockSpec((f, d), lambda vi, ni: (0, 0)),
            pl.BlockSpec((2, d), lambda vi, ni: (0, 0)),
            pl.BlockSpec((2, 1), lambda vi, ni: (0, 0)),
        ],
        out_specs=pl.BlockSpec((None, 2, tm), lambda vi, ni: (vi, 0, ni)),
        compiler_params=pltpu.CompilerParams(
            dimension_semantics=("parallel", "parallel"),
            vmem_limit_bytes=100 * 1024 * 1024),
    )(adj_stack, fh, fl, w_enc.astype(jnp.float32), w2t, bias2)

    e_pad = _round_up(n_edges, 128)
    edges_p = jnp.pad(edge_stack.astype(jnp.int32),
                      ((0, 0), (0, 0), (0, e_pad - n_edges)))

    out = pl.pallas_call(
        _edge_logits_kernel,
        out_shape=jax.ShapeDtypeStruct((n_views, 1, e_pad), jnp.float32),
        grid=(n_views,),
        in_specs=[
            pl.BlockSpec((None, 2, e_pad), lambda vi: (vi, 0, 0)),
            pl.BlockSpec((None, 2, n_nodes), lambda vi: (vi, 0, 0)),
        ],
        out_specs=pl.BlockSpec((None, 1, e_pad), lambda vi: (vi, 0, 0)),
        compiler_params=pltpu.CompilerParams(
            dimension_semantics=("parallel",),
            vmem_limit_bytes=100 * 1024 * 1024),
    )(edges_p, scores)

    logits = out[:, 0, :n_edges][..., None]
    return [logits[i] for i in range(n_views)]
